# baseline (speedup 1.0000x reference)
import functools

import jax
import jax.numpy as jnp
from jax import lax
from jax.experimental import pallas as pl
from jax.experimental.pallas import tpu as pltpu
from jax.experimental.pallas import tpu_sc as plsc

N_NODES = 256
IN_DIM = N_NODES * N_NODES
HIDDEN = 256
N_CELLS = 3
BATCH = 4

K_BLK = 8192
KC = IN_DIM // K_BLK


def _body(x_ref, xt_ref, w_ref, bg_ref, wm_ref, bm_ref, o_ref, acc_ref):
    i = pl.program_id(0)
    k = pl.program_id(1)

    @pl.when(k == 0)
    def _init():
        acc_ref[i] = jnp.zeros((2 * BATCH, HIDDEN), jnp.float32)

    xx = jnp.concatenate(
        [x_ref[:, pl.ds(k * K_BLK, K_BLK)], xt_ref[:, pl.ds(k * K_BLK, K_BLK)]],
        axis=0,
    )
    acc_ref[i] += jnp.dot(
        xx, w_ref[0], preferred_element_type=jnp.float32
    )

    @pl.when((i == N_CELLS - 1) & (k == KC - 1))
    def _epilogue():
        z0 = jnp.maximum(acc_ref[0] + bg_ref[0:1, :], 0.0)
        z1 = jnp.maximum(acc_ref[1] + bg_ref[1:2, :], 0.0) * z0
        z2 = jnp.maximum(acc_ref[2] + bg_ref[2:3, :], 0.0) * z1
        zs = z0 + z1 + z2
        s = jnp.dot(zs, wm_ref[...], preferred_element_type=jnp.float32)
        s = s + 3.0 * bm_ref[...]
        o_ref[...] = s[:BATCH] * s[BATCH:]


_SC_CHUNK = 16384
_SC_ITERS = 12


def _sc_probe(W_flat):
    mesh = plsc.VectorSubcoreMesh(core_axis_name="c", subcore_axis_name="s")

    @functools.partial(
        pl.kernel,
        mesh=mesh,
        out_type=jax.ShapeDtypeStruct((512,), jnp.float32),
        scratch_types=[
            pltpu.VMEM((_SC_CHUNK,), jnp.float32),
            pltpu.SemaphoreType.DMA,
        ],
    )
    def probe(w_hbm, out_hbm, buf, sem):
        wid = lax.axis_index("s") * 2 + lax.axis_index("c")

        def body(j, _):
            base = ((wid + j) % 16) * _SC_CHUNK
            pltpu.async_copy(w_hbm.at[pl.ds(base, _SC_CHUNK)], buf, sem).wait()
            return 0

        lax.fori_loop(0, _SC_ITERS, body, 0)
        pltpu.sync_copy(buf.at[pl.ds(0, 16)], out_hbm.at[pl.ds(wid * 16, 16)])

    return probe(W_flat)


def kernel(flat_adj_matrix, flat_adj_matrix_t, W_gnn, b_gnn, W_mlp, b_mlp):
    bm = b_mlp.reshape(1, N_NODES)
    sc_out = _sc_probe(flat_adj_matrix.reshape(-1))

    grid = (N_CELLS, KC)
    tc_out = pl.pallas_call(
        _body,
        grid=grid,
        in_specs=[
            pl.BlockSpec((BATCH, IN_DIM), lambda i, k: (0, 0)),
            pl.BlockSpec((BATCH, IN_DIM), lambda i, k: (0, 0)),
            pl.BlockSpec((1, K_BLK, HIDDEN), lambda i, k: (i, k, 0)),
            pl.BlockSpec((N_CELLS, HIDDEN), lambda i, k: (0, 0)),
            pl.BlockSpec((HIDDEN, N_NODES), lambda i, k: (0, 0)),
            pl.BlockSpec((1, N_NODES), lambda i, k: (0, 0)),
        ],
        out_specs=pl.BlockSpec((BATCH, N_NODES), lambda i, k: (0, 0)),
        out_shape=jax.ShapeDtypeStruct((BATCH, N_NODES), jnp.float32),
        scratch_shapes=[pltpu.VMEM((N_CELLS, 2 * BATCH, HIDDEN), jnp.float32)],
        compiler_params=pltpu.CompilerParams(
            dimension_semantics=("arbitrary", "arbitrary"),
        ),
    )(flat_adj_matrix, flat_adj_matrix_t, W_gnn, b_gnn, W_mlp, bm)
    return tc_out + jnp.sum(sc_out) * 1e-38


# speedup vs baseline: 1.4756x; 1.4756x over previous
import jax
import jax.numpy as jnp
from jax.experimental import pallas as pl
from jax.experimental.pallas import tpu as pltpu

N_NODES = 256
IN_DIM = N_NODES * N_NODES
HIDDEN = 256
N_CELLS = 3
BATCH = 4

NBUF = 4
CH = 4096
PER_CELL = IN_DIM // CH
CHUNKS = N_CELLS * PER_CELL


def _body(x_ref, xt_ref, w_hbm, bg_ref, wm_ref, bm_ref, o_ref, bufs, acc_ref, sems):
    def copy(c, slot):
        i = c // PER_CELL
        r = c % PER_CELL
        return pltpu.make_async_copy(
            w_hbm.at[i, pl.ds(r * CH, CH), :], bufs.at[slot], sems.at[slot]
        )

    for b in range(NBUF):
        copy(b, b).start()

    acc_ref[...] = jnp.zeros_like(acc_ref)

    for c in range(CHUNKS):
        slot = c % NBUF
        copy(c, slot).wait()
        i = c // PER_CELL
        base = (c % PER_CELL) * CH
        xs = jnp.concatenate(
            [x_ref[:, pl.ds(base, CH)], xt_ref[:, pl.ds(base, CH)]], axis=0
        )
        acc_ref[i] += jnp.dot(xs, bufs[slot], preferred_element_type=jnp.float32)
        if c + NBUF < CHUNKS:
            copy(c + NBUF, slot).start()

    z0 = jnp.maximum(acc_ref[0] + bg_ref[0:1, :], 0.0)
    z1 = jnp.maximum(acc_ref[1] + bg_ref[1:2, :], 0.0) * z0
    z2 = jnp.maximum(acc_ref[2] + bg_ref[2:3, :], 0.0) * z1
    zs = z0 + z1 + z2
    s = jnp.dot(zs, wm_ref[...], preferred_element_type=jnp.float32)
    s = s + 3.0 * bm_ref[...]
    o_ref[...] = s[:BATCH] * s[BATCH:]


def kernel(flat_adj_matrix, flat_adj_matrix_t, W_gnn, b_gnn, W_mlp, b_mlp):
    bm = b_mlp.reshape(1, N_NODES)

    return pl.pallas_call(
        _body,
        in_specs=[
            pl.BlockSpec(memory_space=pltpu.VMEM),
            pl.BlockSpec(memory_space=pltpu.VMEM),
            pl.BlockSpec(memory_space=pltpu.MemorySpace.HBM),
            pl.BlockSpec(memory_space=pltpu.VMEM),
            pl.BlockSpec(memory_space=pltpu.VMEM),
            pl.BlockSpec(memory_space=pltpu.VMEM),
        ],
        out_specs=pl.BlockSpec(memory_space=pltpu.VMEM),
        out_shape=jax.ShapeDtypeStruct((BATCH, N_NODES), jnp.float32),
        scratch_shapes=[
            pltpu.VMEM((NBUF, CH, HIDDEN), jnp.float32),
            pltpu.VMEM((N_CELLS, 2 * BATCH, HIDDEN), jnp.float32),
            pltpu.SemaphoreType.DMA((NBUF,)),
        ],
    )(flat_adj_matrix, flat_adj_matrix_t, W_gnn, b_gnn, W_mlp, bm)


# speedup vs baseline: 1.5203x; 1.0303x over previous
import jax
import jax.numpy as jnp
from jax.experimental import pallas as pl
from jax.experimental.pallas import tpu as pltpu

N_NODES = 256
IN_DIM = N_NODES * N_NODES
HIDDEN = 256
N_CELLS = 3
BATCH = 4

K_BLK = 8192
KC = IN_DIM // K_BLK


def _body(x_ref, xt_ref, w_ref, bg_ref, wm_ref, bm_ref, o_ref, acc_ref):
    i = pl.program_id(0)
    k = pl.program_id(1)

    @pl.when(k == 0)
    def _init():
        acc_ref[i] = jnp.zeros((2 * BATCH, HIDDEN), jnp.float32)

    xx = jnp.concatenate(
        [x_ref[:, pl.ds(k * K_BLK, K_BLK)], xt_ref[:, pl.ds(k * K_BLK, K_BLK)]],
        axis=0,
    )
    acc_ref[i] += jnp.dot(
        xx, w_ref[0], preferred_element_type=jnp.float32
    )

    @pl.when((i == N_CELLS - 1) & (k == KC - 1))
    def _epilogue():
        z0 = jnp.maximum(acc_ref[0] + bg_ref[0:1, :], 0.0)
        z1 = jnp.maximum(acc_ref[1] + bg_ref[1:2, :], 0.0) * z0
        z2 = jnp.maximum(acc_ref[2] + bg_ref[2:3, :], 0.0) * z1
        zs = z0 + z1 + z2
        s = jnp.dot(zs, wm_ref[...], preferred_element_type=jnp.float32)
        s = s + 3.0 * bm_ref[...]
        o_ref[...] = s[:BATCH] * s[BATCH:]


def kernel(flat_adj_matrix, flat_adj_matrix_t, W_gnn, b_gnn, W_mlp, b_mlp):
    bm = b_mlp.reshape(1, N_NODES)

    grid = (N_CELLS, KC)
    return pl.pallas_call(
        _body,
        grid=grid,
        in_specs=[
            pl.BlockSpec((BATCH, IN_DIM), lambda i, k: (0, 0)),
            pl.BlockSpec((BATCH, IN_DIM), lambda i, k: (0, 0)),
            pl.BlockSpec((1, K_BLK, HIDDEN), lambda i, k: (i, k, 0)),
            pl.BlockSpec((N_CELLS, HIDDEN), lambda i, k: (0, 0)),
            pl.BlockSpec((HIDDEN, N_NODES), lambda i, k: (0, 0)),
            pl.BlockSpec((1, N_NODES), lambda i, k: (0, 0)),
        ],
        out_specs=pl.BlockSpec((BATCH, N_NODES), lambda i, k: (0, 0)),
        out_shape=jax.ShapeDtypeStruct((BATCH, N_NODES), jnp.float32),
        scratch_shapes=[pltpu.VMEM((N_CELLS, 2 * BATCH, HIDDEN), jnp.float32)],
        compiler_params=pltpu.CompilerParams(
            dimension_semantics=("arbitrary", "arbitrary"),
        ),
    )(flat_adj_matrix, flat_adj_matrix_t, W_gnn, b_gnn, W_mlp, bm)
